# baseline (speedup 1.0000x reference)
import functools

import jax
import jax.numpy as jnp
from jax import lax
from jax.experimental import pallas as pl
from jax.experimental.pallas import tpu as pltpu
from jax.experimental.pallas import tpu_sc as plsc

VOCAB = 100000
ENT_DIM = 512
OUT_DIM = 128
POS_PAD = 16
RPB = 10000
N_IDS = 32 * 128 * 30

NC, NS = 2, 16
NW = NC * NS
CH = 128
PER_W = N_IDS // NW
NCH = PER_W // CH
IDX_ROWS = 40
IDS_ROWS_PAD = 968


def _proj_body(et_ref, pidx_ref, ptab_ref, w1t_ref, w2t_ref, b_ref, out_ref):
    acc = jnp.dot(et_ref[...], w1t_ref[...], preferred_element_type=jnp.float32)
    pp = jnp.dot(ptab_ref[...], w2t_ref[...], preferred_element_type=jnp.float32)
    pid = pidx_ref[0, 0, :]
    onehot = (pid[:, None] == lax.broadcasted_iota(jnp.int32, (RPB, POS_PAD), 1)
              ).astype(jnp.float32)
    out_ref[...] = acc + jnp.dot(onehot, pp, preferred_element_type=jnp.float32) + b_ref[...]


def _gather_body(proj_hbm, ids_hbm, out_hbm, idx_v, rows0, rows1, sem0, sem1):
    wid = lax.axis_index("s") * NC + lax.axis_index("c")
    start_row = wid * NCH
    slab = pl.multiple_of(8 * (start_row // 8), 8)
    off = start_row - slab
    pltpu.sync_copy(ids_hbm.at[pl.ds(slab, IDX_ROWS)], idx_v)
    out_base = wid * PER_W
    rows = (rows0, rows1)
    sems = (sem0, sem1)

    def start(j, b):
        pltpu.async_copy(proj_hbm.at[idx_v.at[off + j]], rows[b], sems[b])

    def finish(j, b):
        pltpu.make_async_copy(proj_hbm.at[idx_v.at[off + j]], rows[b], sems[b]).wait()
        dst = pl.multiple_of(out_base + j * CH, 8)
        pltpu.sync_copy(rows[b], out_hbm.at[pl.ds(dst, CH)])

    start(0, 0)
    start(1, 1)

    @pl.loop(0, NCH // 2 - 1)
    def _(g):
        j = 2 * g
        finish(j, 0)
        start(j + 2, 0)
        finish(j + 1, 1)
        start(j + 3, 1)

    finish(NCH - 2, 0)
    finish(NCH - 1, 1)


@functools.lru_cache(maxsize=None)
def _make_gather():
    mesh = plsc.VectorSubcoreMesh(
        core_axis_name="c", subcore_axis_name="s", num_cores=NC, num_subcores=NS)
    return pl.kernel(
        _gather_body,
        out_type=jax.ShapeDtypeStruct((N_IDS, OUT_DIM), jnp.float32),
        mesh=mesh,
        scratch_types=[
            pltpu.VMEM((IDX_ROWS, CH), jnp.int32),
            pltpu.VMEM((CH, OUT_DIM), jnp.float32),
            pltpu.VMEM((CH, OUT_DIM), jnp.float32),
            pltpu.SemaphoreType.DMA,
            pltpu.SemaphoreType.DMA,
        ],
    )


def kernel(entity_ids, entity_table, pos_table, entity_id_to_pos_index, W, b):
    w1t = W[:, :ENT_DIM].T
    w2t_pad = jnp.zeros((OUT_DIM, OUT_DIM), jnp.float32).at[
        : W.shape[1] - ENT_DIM, :].set(W[:, ENT_DIM:].T)
    ptab_pad = jnp.zeros((POS_PAD, OUT_DIM), jnp.float32).at[
        :, : pos_table.shape[1]].set(pos_table[:POS_PAD])
    pidx3d = entity_id_to_pos_index.reshape(VOCAB // RPB, 1, RPB).astype(jnp.int32)
    b2d = b.reshape(1, OUT_DIM)

    proj = pl.pallas_call(
        _proj_body,
        grid=(VOCAB // RPB,),
        in_specs=[
            pl.BlockSpec((RPB, ENT_DIM), lambda i: (i, 0)),
            pl.BlockSpec((1, 1, RPB), lambda i: (i, 0, 0)),
            pl.BlockSpec((POS_PAD, OUT_DIM), lambda i: (0, 0)),
            pl.BlockSpec((ENT_DIM, OUT_DIM), lambda i: (0, 0)),
            pl.BlockSpec((OUT_DIM, OUT_DIM), lambda i: (0, 0)),
            pl.BlockSpec((1, OUT_DIM), lambda i: (0, 0)),
        ],
        out_specs=pl.BlockSpec((RPB, OUT_DIM), lambda i: (i, 0)),
        out_shape=jax.ShapeDtypeStruct((VOCAB, OUT_DIM), jnp.float32),
    )(entity_table, pidx3d, ptab_pad, w1t, w2t_pad, b2d)

    nb, nc_, ne = entity_ids.shape
    ids_perm = entity_ids.transpose(0, 2, 1).reshape(-1).astype(jnp.int32)
    ids2d = jnp.pad(ids_perm.reshape(NW * NCH, CH),
                    ((0, IDS_ROWS_PAD - NW * NCH), (0, 0)))
    flat = _make_gather()(proj, ids2d)
    return flat.reshape(nb, ne, nc_, OUT_DIM).transpose(0, 2, 1, 3)


# speedup vs baseline: 1.0095x; 1.0095x over previous
import functools

import jax
import jax.numpy as jnp
from jax import lax
from jax.experimental import pallas as pl
from jax.experimental.pallas import tpu as pltpu
from jax.experimental.pallas import tpu_sc as plsc

VOCAB = 100000
ENT_DIM = 512
OUT_DIM = 128
POS_PAD = 16
RPB = 5000
N_IDS = 32 * 128 * 30

NC, NS = 2, 16
NW = NC * NS
CH = 128
PER_W = N_IDS // NW
NCH = PER_W // CH
IDX_ROWS = 40
IDS_ROWS_PAD = 968


def _proj_body(et_ref, pidx_ref, ptab_ref, w1t_ref, w2t_ref, b_ref, out_ref):
    acc = jnp.dot(et_ref[...], w1t_ref[...], preferred_element_type=jnp.float32)
    pp = jnp.dot(ptab_ref[...], w2t_ref[...], preferred_element_type=jnp.float32)
    pid = pidx_ref[0, 0, :]
    onehot = (pid[:, None] == lax.broadcasted_iota(jnp.int32, (RPB, POS_PAD), 1)
              ).astype(jnp.float32)
    out_ref[...] = acc + jnp.dot(onehot, pp, preferred_element_type=jnp.float32) + b_ref[...]


def _gather_body(proj_hbm, ids_hbm, out_hbm, idx_v,
                 rows0, rows1, rows2, sg0, sg1, sg2, sw0, sw1, sw2):
    wid = lax.axis_index("s") * NC + lax.axis_index("c")
    start_row = wid * NCH
    slab = pl.multiple_of(8 * (start_row // 8), 8)
    off = start_row - slab
    pltpu.sync_copy(ids_hbm.at[pl.ds(slab, IDX_ROWS)], idx_v)
    out_base = wid * PER_W
    rows = (rows0, rows1, rows2)
    sg = (sg0, sg1, sg2)
    sw = (sw0, sw1, sw2)

    def gather_start(j, q):
        pltpu.async_copy(proj_hbm.at[idx_v.at[off + j]], rows[q], sg[q])

    def gather_wait(j, q):
        pltpu.make_async_copy(proj_hbm.at[idx_v.at[off + j]], rows[q], sg[q]).wait()

    def write_start(j, q):
        dst = pl.multiple_of(out_base + j * CH, 8)
        pltpu.async_copy(rows[q], out_hbm.at[pl.ds(dst, CH)], sw[q])

    def write_wait(q):
        pltpu.make_async_copy(rows[q], out_hbm.at[pl.ds(out_base, CH)], sw[q]).wait()

    for q in range(3):
        gather_start(q, q)

    @pl.loop(0, NCH // 3 - 1)
    def _(g):
        j0 = 3 * g
        for q in range(3):
            j = j0 + q
            gather_wait(j, q)
            write_start(j, q)
            write_wait(q)
            gather_start(j + 3, q)

    for q in range(3):
        j = NCH - 3 + q
        gather_wait(j, q)
        write_start(j, q)
    for q in range(3):
        write_wait(q)


@functools.lru_cache(maxsize=None)
def _make_gather():
    mesh = plsc.VectorSubcoreMesh(
        core_axis_name="c", subcore_axis_name="s", num_cores=NC, num_subcores=NS)
    return pl.kernel(
        _gather_body,
        out_type=jax.ShapeDtypeStruct((N_IDS, OUT_DIM), jnp.float32),
        mesh=mesh,
        scratch_types=[
            pltpu.VMEM((IDX_ROWS, CH), jnp.int32),
            pltpu.VMEM((CH, OUT_DIM), jnp.float32),
            pltpu.VMEM((CH, OUT_DIM), jnp.float32),
            pltpu.VMEM((CH, OUT_DIM), jnp.float32),
            pltpu.SemaphoreType.DMA,
            pltpu.SemaphoreType.DMA,
            pltpu.SemaphoreType.DMA,
            pltpu.SemaphoreType.DMA,
            pltpu.SemaphoreType.DMA,
            pltpu.SemaphoreType.DMA,
        ],
    )


def kernel(entity_ids, entity_table, pos_table, entity_id_to_pos_index, W, b):
    w1t = W[:, :ENT_DIM].T
    w2t_pad = jnp.zeros((OUT_DIM, OUT_DIM), jnp.float32).at[
        : W.shape[1] - ENT_DIM, :].set(W[:, ENT_DIM:].T)
    ptab_pad = jnp.zeros((POS_PAD, OUT_DIM), jnp.float32).at[
        :, : pos_table.shape[1]].set(pos_table[:POS_PAD])
    pidx3d = entity_id_to_pos_index.reshape(VOCAB // RPB, 1, RPB).astype(jnp.int32)
    b2d = b.reshape(1, OUT_DIM)

    proj = pl.pallas_call(
        _proj_body,
        grid=(VOCAB // RPB,),
        in_specs=[
            pl.BlockSpec((RPB, ENT_DIM), lambda i: (i, 0)),
            pl.BlockSpec((1, 1, RPB), lambda i: (i, 0, 0)),
            pl.BlockSpec((POS_PAD, OUT_DIM), lambda i: (0, 0)),
            pl.BlockSpec((ENT_DIM, OUT_DIM), lambda i: (0, 0)),
            pl.BlockSpec((OUT_DIM, OUT_DIM), lambda i: (0, 0)),
            pl.BlockSpec((1, OUT_DIM), lambda i: (0, 0)),
        ],
        out_specs=pl.BlockSpec((RPB, OUT_DIM), lambda i: (i, 0)),
        out_shape=jax.ShapeDtypeStruct((VOCAB, OUT_DIM), jnp.float32),
    )(entity_table, pidx3d, ptab_pad, w1t, w2t_pad, b2d)

    nb, nc_, ne = entity_ids.shape
    ids_perm = entity_ids.transpose(0, 2, 1).reshape(-1).astype(jnp.int32)
    ids2d = jnp.pad(ids_perm.reshape(NW * NCH, CH),
                    ((0, IDS_ROWS_PAD - NW * NCH), (0, 0)))
    flat = _make_gather()(proj, ids2d)
    return flat.reshape(nb, ne, nc_, OUT_DIM).transpose(0, 2, 1, 3)
